# Initial kernel scaffold; baseline (speedup 1.0000x reference)
import jax
import jax.numpy as jnp
from jax.experimental import pallas as pl


def kernel(label, edge_index, weight, table, W1, b1, W2, b2):
    raise NotImplementedError("write your pallas kernel here")


# speedup vs baseline: 10.6102x; 10.6102x over previous
import dataclasses
import functools

import jax
import jax.numpy as jnp
from jax import lax
from jax.experimental import pallas as pl
from jax.experimental.pallas import tpu as pltpu
from jax.experimental.pallas import tpu_sc as plsc

_NC = 2
_NS = 16
_NW = _NC * _NS
_L = 16
_CH = 128
_D = 128


def _sc_mesh():
    return plsc.VectorSubcoreMesh(core_axis_name="c", subcore_axis_name="s")


def _sc_params():
    cp = pltpu.CompilerParams()
    if "needs_layout_passes" in pltpu.CompilerParams.__dataclass_fields__:
        cp = dataclasses.replace(cp, needs_layout_passes=False)
    return cp


def _sc_degree(dE, wE, n_acc, n_chunks):
    m_pad = dE.shape[0]
    per_w = m_pad // _NW
    rows_per_tile = n_acc // _NS
    zsteps = rows_per_tile // _CH

    @functools.partial(
        pl.kernel,
        out_type=jax.ShapeDtypeStruct((_NC, n_acc), jnp.float32),
        mesh=_sc_mesh(),
        scratch_types=[
            pltpu.VMEM((_CH,), jnp.int32),
            pltpu.VMEM((_CH,), jnp.float32),
            pltpu.VMEM((_CH,), jnp.float32),
            pltpu.VMEM_SHARED((n_acc,), jnp.float32),
        ],
        compiler_params=_sc_params(),
    )
    def k(d_hbm, w_hbm, out_hbm, d_v, w_v, z_v, acc):
        cid = lax.axis_index("c")
        sid = lax.axis_index("s")
        wid = cid * _NS + sid
        zero16 = jnp.zeros((_L,), jnp.float32)

        @pl.loop(0, _CH, step=_L)
        def _(r):
            z_v[pl.ds(r, _L)] = zero16

        row0 = sid * rows_per_tile

        @pl.loop(0, zsteps)
        def _(i):
            pltpu.sync_copy(z_v, acc.at[pl.ds(row0 + i * _CH, _CH)])

        plsc.subcore_barrier()

        base = wid * per_w

        @pl.loop(0, n_chunks)
        def _(ci):
            off = base + ci * _CH
            pltpu.sync_copy(d_hbm.at[pl.ds(off, _CH)], d_v)
            pltpu.sync_copy(w_hbm.at[pl.ds(off, _CH)], w_v)
            pltpu.sync_copy(w_v, acc.at[d_v], add=True)

        plsc.subcore_barrier()

        @pl.loop(0, zsteps)
        def _(i):
            r0 = row0 + i * _CH
            pltpu.sync_copy(acc.at[pl.ds(r0, _CH)],
                            out_hbm.at[cid, pl.ds(r0, _CH)])

    return k(dE, wE)


def _sc_msg_pass(t, map_arr, dinv, sE, dE, wE, n_acc, n_chunks):
    m_pad = sE.shape[0]
    per_w = m_pad // _NW
    n = map_arr.shape[0]
    rows_per_tile = n_acc // _NS
    zsteps = rows_per_tile // _CH

    @functools.partial(
        pl.kernel,
        out_type=jax.ShapeDtypeStruct((_NC, n_acc, _D), jnp.float32),
        mesh=_sc_mesh(),
        scratch_types=[
            pltpu.VMEM((n,), jnp.int32),
            pltpu.VMEM((n,), jnp.float32),
            pltpu.VMEM((_CH,), jnp.int32),
            pltpu.VMEM((_CH,), jnp.int32),
            pltpu.VMEM((_CH,), jnp.float32),
            pltpu.VMEM((_CH,), jnp.int32),
            pltpu.VMEM((_CH,), jnp.float32),
            pltpu.VMEM((_CH, _D), jnp.float32),
            pltpu.VMEM_SHARED((n_acc, _D), jnp.float32),
            pltpu.SemaphoreType.DMA,
        ],
        compiler_params=_sc_params(),
    )
    def k(t_hbm, map_hbm, dinv_hbm, s_hbm, d_hbm, w_hbm, out_hbm,
          map_v, dinv_v, s_v, d_v, w_v, ls_v, c_v, rows_v, acc, sem):
        cid = lax.axis_index("c")
        sid = lax.axis_index("s")
        wid = cid * _NS + sid

        pltpu.sync_copy(map_hbm, map_v)
        pltpu.sync_copy(dinv_hbm, dinv_v)

        zero16 = jnp.zeros((_L,), jnp.float32)

        @pl.loop(0, _CH)
        def _(r):
            @pl.loop(0, _D, step=_L)
            def _(cc):
                rows_v[r, pl.ds(cc, _L)] = zero16

        row0 = sid * rows_per_tile

        @pl.loop(0, zsteps)
        def _(i):
            pltpu.sync_copy(rows_v, acc.at[pl.ds(row0 + i * _CH, _CH)])

        plsc.subcore_barrier()

        base = wid * per_w

        @pl.loop(0, n_chunks)
        def _(ci):
            off = base + ci * _CH
            pltpu.sync_copy(s_hbm.at[pl.ds(off, _CH)], s_v)
            pltpu.sync_copy(d_hbm.at[pl.ds(off, _CH)], d_v)
            pltpu.sync_copy(w_hbm.at[pl.ds(off, _CH)], w_v)

            @pl.loop(0, _CH, step=_L)
            def _(kk):
                s16 = s_v[pl.ds(kk, _L)]
                ls_v[pl.ds(kk, _L)] = plsc.load_gather(map_v, [s16])
                c_v[pl.ds(kk, _L)] = w_v[pl.ds(kk, _L)] * plsc.load_gather(
                    dinv_v, [s16])

            pltpu.async_copy(t_hbm.at[ls_v], rows_v, sem).wait()

            @pl.loop(0, _CH)
            def _(e):
                cb = plsc.load_gather(c_v, [jnp.full((_L,), 0, jnp.int32) + e])
                for j in range(_D // _L):
                    sl = pl.ds(j * _L, _L)
                    rows_v[e, sl] = rows_v[e, sl] * cb

            pltpu.sync_copy(rows_v, acc.at[d_v], add=True)

        plsc.subcore_barrier()

        @pl.loop(0, zsteps)
        def _(i):
            r0 = row0 + i * _CH
            pltpu.sync_copy(acc.at[pl.ds(r0, _CH)],
                            out_hbm.at[cid, pl.ds(r0, _CH)])

    return k(t, map_arr, dinv, sE, dE, wE)


def _tc_matmul(x, w):
    m, kdim = x.shape
    blk = 2000

    def body(x_ref, w_ref, o_ref):
        o_ref[...] = jnp.dot(x_ref[...], w_ref[...],
                             preferred_element_type=jnp.float32)

    return pl.pallas_call(
        body,
        grid=(m // blk,),
        in_specs=[
            pl.BlockSpec((blk, kdim), lambda i: (i, 0)),
            pl.BlockSpec((kdim, w.shape[1]), lambda i: (0, 0)),
        ],
        out_specs=pl.BlockSpec((blk, w.shape[1]), lambda i: (i, 0)),
        out_shape=jax.ShapeDtypeStruct((m, w.shape[1]), jnp.float32),
    )(x, w)


def _tc_dinv(degp):
    n = degp.shape[1]

    def body(p_ref, o_ref):
        deg = p_ref[0:1, :] + p_ref[1:2, :]
        o_ref[...] = jnp.where(deg > 0.0, lax.rsqrt(jnp.maximum(deg, 1e-30)),
                               0.0)

    return pl.pallas_call(
        body,
        out_shape=jax.ShapeDtypeStruct((1, n), jnp.float32),
    )(degp)


def _tc_hidden(p, dinv2d, b1, w2):
    n = p.shape[1]
    blk = 2048

    def body(p_ref, dinv_ref, b_ref, w_ref, o_ref):
        h = jnp.maximum(dinv_ref[...] * (p_ref[0] + p_ref[1]) + b_ref[...],
                        0.0)
        o_ref[...] = jnp.dot(h, w_ref[...], preferred_element_type=jnp.float32)

    return pl.pallas_call(
        body,
        grid=(n // blk,),
        in_specs=[
            pl.BlockSpec((_NC, blk, _D), lambda i: (0, i, 0)),
            pl.BlockSpec((blk, 1), lambda i: (i, 0)),
            pl.BlockSpec((1, _D), lambda i: (0, 0)),
            pl.BlockSpec((_D, _D), lambda i: (0, 0)),
        ],
        out_specs=pl.BlockSpec((blk, _D), lambda i: (i, 0)),
        out_shape=jax.ShapeDtypeStruct((n, _D), jnp.float32),
    )(p, dinv2d, b1, w2)


def _tc_final(q, dinv2d, b2):
    n = q.shape[1]
    blk = 2048

    def body(q_ref, dinv_ref, b_ref, o_ref):
        o_ref[...] = dinv_ref[...] * (q_ref[0] + q_ref[1]) + b_ref[...]

    return pl.pallas_call(
        body,
        grid=(n // blk,),
        in_specs=[
            pl.BlockSpec((_NC, blk, _D), lambda i: (0, i, 0)),
            pl.BlockSpec((blk, 1), lambda i: (i, 0)),
            pl.BlockSpec((1, _D), lambda i: (0, 0)),
        ],
        out_specs=pl.BlockSpec((blk, _D), lambda i: (i, 0)),
        out_shape=jax.ShapeDtypeStruct((n, _D), jnp.float32),
    )(q, dinv2d, b2)


def kernel(label, edge_index, weight, table, W1, b1, W2, b2):
    n = label.shape[0]
    e = weight.shape[0]
    m = e + n
    n_acc = -(-n // (_NS * _CH)) * (_NS * _CH)

    chunk_items = _NW * _CH
    n_chunks = -(-m // chunk_items)
    m_pad = n_chunks * chunk_items
    pad = m_pad - m

    src = edge_index[0].astype(jnp.int32)
    dst = edge_index[1].astype(jnp.int32)
    loop = jnp.arange(n, dtype=jnp.int32)
    padr = jnp.arange(pad, dtype=jnp.int32) % n
    sE = jnp.concatenate([src, loop, padr])
    dE = jnp.concatenate([dst, loop, padr])
    wE = jnp.concatenate([weight.astype(jnp.float32),
                          jnp.ones((n,), jnp.float32),
                          jnp.zeros((pad,), jnp.float32)])

    degp = _sc_degree(dE, wE, n_acc, n_chunks)
    t1 = _tc_matmul(table, W1)
    dinv_row = _tc_dinv(degp)
    dinv = dinv_row[0, :n]
    dinv2d_pad = dinv_row.reshape((n_acc, 1))

    p = _sc_msg_pass(t1, label.astype(jnp.int32), dinv, sE, dE, wE, n_acc,
                     n_chunks)
    t2 = _tc_hidden(p, dinv2d_pad, b1.reshape((1, _D)), W2)
    q = _sc_msg_pass(t2, loop, dinv, sE, dE, wE, n_acc, n_chunks)
    return _tc_final(q, dinv2d_pad, b2.reshape((1, _D)))[:n]


# speedup vs baseline: 13.5678x; 1.2787x over previous
import dataclasses
import functools

import jax
import jax.numpy as jnp
from jax import lax
from jax.experimental import pallas as pl
from jax.experimental.pallas import tpu as pltpu
from jax.experimental.pallas import tpu_sc as plsc

_NC = 2
_NS = 16
_NW = _NC * _NS
_L = 16
_CH = 128
_CHM = 64
_D = 128


def _sc_mesh():
    return plsc.VectorSubcoreMesh(core_axis_name="c", subcore_axis_name="s")


def _sc_params():
    cp = pltpu.CompilerParams()
    if "needs_layout_passes" in pltpu.CompilerParams.__dataclass_fields__:
        cp = dataclasses.replace(cp, needs_layout_passes=False)
    return cp


def _sc_degree(dE, wE, n_acc, n_chunks):
    m_pad = dE.shape[0]
    per_w = m_pad // _NW
    rows_per_tile = n_acc // _NS
    zsteps = rows_per_tile // _CH

    @functools.partial(
        pl.kernel,
        out_type=jax.ShapeDtypeStruct((_NC, n_acc), jnp.float32),
        mesh=_sc_mesh(),
        scratch_types=[
            pltpu.VMEM((_CH,), jnp.int32),
            pltpu.VMEM((_CH,), jnp.float32),
            pltpu.VMEM((_CH,), jnp.float32),
            pltpu.VMEM_SHARED((n_acc,), jnp.float32),
        ],
        compiler_params=_sc_params(),
    )
    def k(d_hbm, w_hbm, out_hbm, d_v, w_v, z_v, acc):
        cid = lax.axis_index("c")
        sid = lax.axis_index("s")
        wid = cid * _NS + sid
        zero16 = jnp.zeros((_L,), jnp.float32)

        @pl.loop(0, _CH, step=_L)
        def _(r):
            z_v[pl.ds(r, _L)] = zero16

        row0 = sid * rows_per_tile

        @pl.loop(0, zsteps)
        def _(i):
            pltpu.sync_copy(z_v, acc.at[pl.ds(row0 + i * _CH, _CH)])

        plsc.subcore_barrier()

        base = wid * per_w

        @pl.loop(0, n_chunks)
        def _(ci):
            off = base + ci * _CH
            pltpu.sync_copy(d_hbm.at[pl.ds(off, _CH)], d_v)
            pltpu.sync_copy(w_hbm.at[pl.ds(off, _CH)], w_v)
            pltpu.sync_copy(w_v, acc.at[d_v], add=True)

        plsc.subcore_barrier()

        @pl.loop(0, zsteps)
        def _(i):
            r0 = row0 + i * _CH
            pltpu.sync_copy(acc.at[pl.ds(r0, _CH)],
                            out_hbm.at[cid, pl.ds(r0, _CH)])

    return k(dE, wE)


def _sc_msg_pass(t, map_arr, dinv, sE, dE, wE, n_acc, n_chunks):
    m_pad = sE.shape[0]
    per_w = m_pad // _NW
    n = map_arr.shape[0]
    rows_per_tile = n_acc // _NS
    assert n_chunks % 4 == 0 and n_chunks >= 8

    @functools.partial(
        pl.kernel,
        out_type=jax.ShapeDtypeStruct((_NC, n_acc, _D), jnp.float32),
        mesh=_sc_mesh(),
        scratch_types=(
            [pltpu.VMEM((n,), jnp.int32),
             pltpu.VMEM((n,), jnp.float32)]
            + [pltpu.VMEM((_CHM,), jnp.int32) for _ in range(4)]
            + [pltpu.VMEM((_CHM,), jnp.int32) for _ in range(4)]
            + [pltpu.VMEM((_CHM,), jnp.float32) for _ in range(4)]
            + [pltpu.VMEM((_CHM,), jnp.int32) for _ in range(4)]
            + [pltpu.VMEM((_CHM,), jnp.float32) for _ in range(4)]
            + [pltpu.VMEM((_CHM, _D), jnp.float32) for _ in range(2)]
            + [pltpu.VMEM_SHARED((n_acc, _D), jnp.float32)]
            + [pltpu.SemaphoreType.DMA for _ in range(8)]
        ),
        compiler_params=_sc_params(),
    )
    def k(t_hbm, map_hbm, dinv_hbm, s_hbm, d_hbm, w_hbm, out_hbm, *sc):
        map_v, dinv_v = sc[0], sc[1]
        svs, dvs, wvs = sc[2:6], sc[6:10], sc[10:14]
        lvs, cvs = sc[14:18], sc[18:22]
        rows = sc[22:24]
        acc = sc[24]
        isems, gsems, ssems = sc[25:29], sc[29:31], sc[31:33]

        cid = lax.axis_index("c")
        sid = lax.axis_index("s")
        wid = cid * _NS + sid
        base = wid * per_w
        row0 = sid * rows_per_tile

        pltpu.sync_copy(map_hbm, map_v)
        pltpu.sync_copy(dinv_hbm, dinv_v)

        zero16 = jnp.zeros((_L,), jnp.float32)

        @pl.loop(0, _CHM)
        def _(r):
            @pl.loop(0, _D, step=_L)
            def _(cc):
                rows[0][r, pl.ds(cc, _L)] = zero16

        @pl.loop(0, rows_per_tile, step=_CHM)
        def _(z):
            pltpu.sync_copy(rows[0], acc.at[pl.ds(row0 + z, _CHM)])

        plsc.subcore_barrier()

        def idx_start(i, q):
            off = base + i * _CHM
            pltpu.async_copy(s_hbm.at[pl.ds(off, _CHM)], svs[q], isems[q])
            pltpu.async_copy(d_hbm.at[pl.ds(off, _CHM)], dvs[q], isems[q])
            pltpu.async_copy(w_hbm.at[pl.ds(off, _CHM)], wvs[q], isems[q])

        def idx_wait(i, q):
            off = base + i * _CHM
            pltpu.make_async_copy(s_hbm.at[pl.ds(off, _CHM)], svs[q],
                                  isems[q]).wait()
            pltpu.make_async_copy(d_hbm.at[pl.ds(off, _CHM)], dvs[q],
                                  isems[q]).wait()
            pltpu.make_async_copy(w_hbm.at[pl.ds(off, _CHM)], wvs[q],
                                  isems[q]).wait()

        def compute_lc(q):
            @pl.loop(0, _CHM, step=_L)
            def _(kk):
                s16 = svs[q][pl.ds(kk, _L)]
                lvs[q][pl.ds(kk, _L)] = plsc.load_gather(map_v, [s16])
                cvs[q][pl.ds(kk, _L)] = wvs[q][pl.ds(kk, _L)] * (
                    plsc.load_gather(dinv_v, [s16]))

        def gather_start(q, b):
            pltpu.async_copy(t_hbm.at[lvs[q]], rows[b], gsems[b])

        def gather_wait(q, b):
            pltpu.make_async_copy(t_hbm.at[lvs[q]], rows[b], gsems[b]).wait()

        def scatter_start(q, b):
            pltpu.async_copy(rows[b], acc.at[dvs[q]], ssems[b], add=True)

        def scatter_wait(q, b):
            pltpu.make_async_copy(rows[b], acc.at[dvs[q]], ssems[b]).wait()

        idx_start(0, 0)
        idx_start(1, 1)
        idx_start(2, 2)
        idx_wait(0, 0)
        compute_lc(0)
        gather_start(0, 0)

        @pl.loop(0, n_chunks, step=4)
        def _(g):
            for j in range(4):
                q, b = j, j & 1
                q2, b2 = (j + 1) & 3, 1 - (j & 1)
                i = g + j

                gather_wait(q, b)

                @pl.loop(0, _CHM)
                def _(e):
                    cb = plsc.load_gather(
                        cvs[q], [jnp.full((_L,), 0, jnp.int32) + e])
                    for jj in range(_D // _L):
                        sl = pl.ds(jj * _L, _L)
                        rows[b][e, sl] = rows[b][e, sl] * cb

                scatter_start(q, b)

                @pl.when(i + 1 < n_chunks)
                def _():
                    @pl.when(i >= 1)
                    def _():
                        scatter_wait((j + 3) & 3, b2)

                    idx_wait(i + 1, q2)
                    compute_lc(q2)
                    gather_start(q2, b2)

                    @pl.when(i + 3 < n_chunks)
                    def _():
                        idx_start(i + 3, (j + 3) & 3)

        scatter_wait(2, 0)
        scatter_wait(3, 1)

        plsc.subcore_barrier()

        @pl.loop(0, rows_per_tile, step=_CH)
        def _(z):
            r0 = row0 + z
            pltpu.sync_copy(acc.at[pl.ds(r0, _CH)],
                            out_hbm.at[cid, pl.ds(r0, _CH)])

    return k(t, map_arr, dinv, sE, dE, wE)


def _tc_matmul(x, w):
    m, kdim = x.shape
    blk = 2000

    def body(x_ref, w_ref, o_ref):
        o_ref[...] = jnp.dot(x_ref[...], w_ref[...],
                             preferred_element_type=jnp.float32)

    return pl.pallas_call(
        body,
        grid=(m // blk,),
        in_specs=[
            pl.BlockSpec((blk, kdim), lambda i: (i, 0)),
            pl.BlockSpec((kdim, w.shape[1]), lambda i: (0, 0)),
        ],
        out_specs=pl.BlockSpec((blk, w.shape[1]), lambda i: (i, 0)),
        out_shape=jax.ShapeDtypeStruct((m, w.shape[1]), jnp.float32),
    )(x, w)


def _tc_dinv(degp):
    n = degp.shape[1]

    def body(p_ref, o_ref):
        deg = p_ref[0:1, :] + p_ref[1:2, :]
        o_ref[...] = jnp.where(deg > 0.0, lax.rsqrt(jnp.maximum(deg, 1e-30)),
                               0.0)

    return pl.pallas_call(
        body,
        out_shape=jax.ShapeDtypeStruct((1, n), jnp.float32),
    )(degp)


def _tc_hidden(p, dinv2d, b1, w2):
    n = p.shape[1]
    blk = 2048

    def body(p_ref, dinv_ref, b_ref, w_ref, o_ref):
        h = jnp.maximum(dinv_ref[...] * (p_ref[0] + p_ref[1]) + b_ref[...],
                        0.0)
        o_ref[...] = jnp.dot(h, w_ref[...], preferred_element_type=jnp.float32)

    return pl.pallas_call(
        body,
        grid=(n // blk,),
        in_specs=[
            pl.BlockSpec((_NC, blk, _D), lambda i: (0, i, 0)),
            pl.BlockSpec((blk, 1), lambda i: (i, 0)),
            pl.BlockSpec((1, _D), lambda i: (0, 0)),
            pl.BlockSpec((_D, _D), lambda i: (0, 0)),
        ],
        out_specs=pl.BlockSpec((blk, _D), lambda i: (i, 0)),
        out_shape=jax.ShapeDtypeStruct((n, _D), jnp.float32),
    )(p, dinv2d, b1, w2)


def _tc_final(q, dinv2d, b2):
    n = q.shape[1]
    blk = 2048

    def body(q_ref, dinv_ref, b_ref, o_ref):
        o_ref[...] = dinv_ref[...] * (q_ref[0] + q_ref[1]) + b_ref[...]

    return pl.pallas_call(
        body,
        grid=(n // blk,),
        in_specs=[
            pl.BlockSpec((_NC, blk, _D), lambda i: (0, i, 0)),
            pl.BlockSpec((blk, 1), lambda i: (i, 0)),
            pl.BlockSpec((1, _D), lambda i: (0, 0)),
        ],
        out_specs=pl.BlockSpec((blk, _D), lambda i: (i, 0)),
        out_shape=jax.ShapeDtypeStruct((n, _D), jnp.float32),
    )(q, dinv2d, b2)


def kernel(label, edge_index, weight, table, W1, b1, W2, b2):
    n = label.shape[0]
    e = weight.shape[0]
    m = e + n
    n_acc = -(-n // (_NS * _CH)) * (_NS * _CH)

    align = _NW * _CHM * 4
    m_pad = -(-m // align) * align
    pad = m_pad - m
    n_chunks_deg = m_pad // (_NW * _CH)
    n_chunks_msg = m_pad // (_NW * _CHM)

    src = edge_index[0].astype(jnp.int32)
    dst = edge_index[1].astype(jnp.int32)
    loop = jnp.arange(n, dtype=jnp.int32)
    padr = jnp.arange(pad, dtype=jnp.int32) % n
    sE = jnp.concatenate([src, loop, padr])
    dE = jnp.concatenate([dst, loop, padr])
    wE = jnp.concatenate([weight.astype(jnp.float32),
                          jnp.ones((n,), jnp.float32),
                          jnp.zeros((pad,), jnp.float32)])

    degp = _sc_degree(dE, wE, n_acc, n_chunks_deg)
    t1 = _tc_matmul(table, W1)
    dinv_row = _tc_dinv(degp)
    dinv = dinv_row[0, :n]
    dinv2d_pad = dinv_row.reshape((n_acc, 1))

    p = _sc_msg_pass(t1, label.astype(jnp.int32), dinv, sE, dE, wE, n_acc,
                     n_chunks_msg)
    t2 = _tc_hidden(p, dinv2d_pad, b1.reshape((1, _D)), W2)
    q = _sc_msg_pass(t2, loop, dinv, sE, dE, wE, n_acc, n_chunks_msg)
    return _tc_final(q, dinv2d_pad, b2.reshape((1, _D)))[:n]


# speedup vs baseline: 15.7933x; 1.1640x over previous
import dataclasses
import functools

import jax
import jax.numpy as jnp
from jax import lax
from jax.experimental import pallas as pl
from jax.experimental.pallas import tpu as pltpu
from jax.experimental.pallas import tpu_sc as plsc

_NC = 2
_NS = 16
_NW = _NC * _NS
_L = 16
_CH = 128
_CHM = 64
_D = 128


def _sc_mesh():
    return plsc.VectorSubcoreMesh(core_axis_name="c", subcore_axis_name="s")


def _sc_params():
    cp = pltpu.CompilerParams()
    if "needs_layout_passes" in pltpu.CompilerParams.__dataclass_fields__:
        cp = dataclasses.replace(cp, needs_layout_passes=False)
    return cp


def _sc_degree(dE, wE, n_acc, n_chunks):
    m_pad = dE.shape[0]
    per_w = m_pad // _NW
    rows_per_tile = n_acc // _NS
    zsteps = rows_per_tile // _CH

    @functools.partial(
        pl.kernel,
        out_type=jax.ShapeDtypeStruct((_NC, n_acc), jnp.float32),
        mesh=_sc_mesh(),
        scratch_types=[
            pltpu.VMEM((_CH,), jnp.int32),
            pltpu.VMEM((_CH,), jnp.float32),
            pltpu.VMEM((_CH,), jnp.float32),
            pltpu.VMEM_SHARED((n_acc,), jnp.float32),
        ],
        compiler_params=_sc_params(),
    )
    def k(d_hbm, w_hbm, out_hbm, d_v, w_v, z_v, acc):
        cid = lax.axis_index("c")
        sid = lax.axis_index("s")
        wid = cid * _NS + sid
        zero16 = jnp.zeros((_L,), jnp.float32)

        @pl.loop(0, _CH, step=_L)
        def _(r):
            z_v[pl.ds(r, _L)] = zero16

        row0 = sid * rows_per_tile

        @pl.loop(0, zsteps)
        def _(i):
            pltpu.sync_copy(z_v, acc.at[pl.ds(row0 + i * _CH, _CH)])

        plsc.subcore_barrier()

        base = wid * per_w

        @pl.loop(0, n_chunks)
        def _(ci):
            off = base + ci * _CH
            pltpu.sync_copy(d_hbm.at[pl.ds(off, _CH)], d_v)
            pltpu.sync_copy(w_hbm.at[pl.ds(off, _CH)], w_v)
            pltpu.sync_copy(w_v, acc.at[d_v], add=True)

        plsc.subcore_barrier()

        @pl.loop(0, zsteps)
        def _(i):
            r0 = row0 + i * _CH
            pltpu.sync_copy(acc.at[pl.ds(r0, _CH)],
                            out_hbm.at[cid, pl.ds(r0, _CH)])

    return k(dE, wE)


def _sc_msg_pass(t, map_arr, dinv, sE, dE, wE, n_acc, n_chunks):
    m_pad = sE.shape[0]
    per_w = m_pad // _NW
    n = map_arr.shape[0]
    rows_per_tile = n_acc // _NS
    assert n_chunks % 4 == 0 and n_chunks >= 8

    @functools.partial(
        pl.kernel,
        out_type=jax.ShapeDtypeStruct((_NC, n_acc, _D), jnp.float32),
        mesh=_sc_mesh(),
        scratch_types=(
            [pltpu.VMEM((n,), jnp.int32),
             pltpu.VMEM((n,), jnp.float32)]
            + [pltpu.VMEM((_CHM,), jnp.int32) for _ in range(4)]
            + [pltpu.VMEM((_CHM,), jnp.int32) for _ in range(4)]
            + [pltpu.VMEM((_CHM,), jnp.float32) for _ in range(4)]
            + [pltpu.VMEM((_CHM,), jnp.int32) for _ in range(4)]
            + [pltpu.VMEM((_CHM,), jnp.float32) for _ in range(4)]
            + [pltpu.VMEM((_CHM, _D), jnp.float32) for _ in range(2)]
            + [pltpu.VMEM_SHARED((n_acc, _D), jnp.float32)]
            + [pltpu.SemaphoreType.DMA for _ in range(8)]
        ),
        compiler_params=_sc_params(),
    )
    def k(t_hbm, map_hbm, dinv_hbm, s_hbm, d_hbm, w_hbm, out_hbm, *sc):
        map_v, dinv_v = sc[0], sc[1]
        svs, dvs, wvs = sc[2:6], sc[6:10], sc[10:14]
        lvs, cvs = sc[14:18], sc[18:22]
        rows = sc[22:24]
        acc = sc[24]
        isems, gsems, ssems = sc[25:29], sc[29:31], sc[31:33]

        cid = lax.axis_index("c")
        sid = lax.axis_index("s")
        wid = cid * _NS + sid
        base = wid * per_w
        row0 = sid * rows_per_tile

        pltpu.sync_copy(map_hbm, map_v)
        pltpu.sync_copy(dinv_hbm, dinv_v)

        zero16 = jnp.zeros((_L,), jnp.float32)

        @pl.loop(0, _CHM)
        def _(r):
            @pl.loop(0, _D, step=_L)
            def _(cc):
                rows[0][r, pl.ds(cc, _L)] = zero16

        @pl.loop(0, rows_per_tile, step=_CHM)
        def _(z):
            pltpu.sync_copy(rows[0], acc.at[pl.ds(row0 + z, _CHM)])

        plsc.subcore_barrier()

        def idx_start(i, q):
            off = base + i * _CHM
            pltpu.async_copy(s_hbm.at[pl.ds(off, _CHM)], svs[q], isems[q])
            pltpu.async_copy(d_hbm.at[pl.ds(off, _CHM)], dvs[q], isems[q])
            pltpu.async_copy(w_hbm.at[pl.ds(off, _CHM)], wvs[q], isems[q])

        def idx_wait(i, q):
            off = base + i * _CHM
            pltpu.make_async_copy(s_hbm.at[pl.ds(off, _CHM)], svs[q],
                                  isems[q]).wait()
            pltpu.make_async_copy(d_hbm.at[pl.ds(off, _CHM)], dvs[q],
                                  isems[q]).wait()
            pltpu.make_async_copy(w_hbm.at[pl.ds(off, _CHM)], wvs[q],
                                  isems[q]).wait()

        def compute_lc(q):
            @plsc.parallel_loop(0, _CHM, step=_L, unroll=2)
            def _(kk):
                s16 = svs[q][pl.ds(kk, _L)]
                lvs[q][pl.ds(kk, _L)] = plsc.load_gather(map_v, [s16])
                cvs[q][pl.ds(kk, _L)] = wvs[q][pl.ds(kk, _L)] * (
                    plsc.load_gather(dinv_v, [s16]))

        def gather_start(q, b):
            pltpu.async_copy(t_hbm.at[lvs[q]], rows[b], gsems[b])

        def gather_wait(q, b):
            pltpu.make_async_copy(t_hbm.at[lvs[q]], rows[b], gsems[b]).wait()

        def scatter_start(q, b):
            pltpu.async_copy(rows[b], acc.at[dvs[q]], ssems[b], add=True)

        def scatter_wait(q, b):
            pltpu.make_async_copy(rows[b], acc.at[dvs[q]], ssems[b]).wait()

        idx_start(0, 0)
        idx_start(1, 1)
        idx_start(2, 2)
        idx_wait(0, 0)
        compute_lc(0)
        gather_start(0, 0)

        @pl.loop(0, n_chunks, step=4)
        def _(g):
            for j in range(4):
                q, b = j, j & 1
                q2, b2 = (j + 1) & 3, 1 - (j & 1)
                i = g + j

                gather_wait(q, b)

                @plsc.parallel_loop(0, _CHM, unroll=4)
                def _(e):
                    cb = plsc.load_gather(
                        cvs[q], [jnp.full((_L,), 0, jnp.int32) + e])
                    for jj in range(_D // _L):
                        sl = pl.ds(jj * _L, _L)
                        rows[b][e, sl] = rows[b][e, sl] * cb

                scatter_start(q, b)

                @pl.when(i + 1 < n_chunks)
                def _():
                    @pl.when(i >= 1)
                    def _():
                        scatter_wait((j + 3) & 3, b2)

                    idx_wait(i + 1, q2)
                    compute_lc(q2)
                    gather_start(q2, b2)

                    @pl.when(i + 3 < n_chunks)
                    def _():
                        idx_start(i + 3, (j + 3) & 3)

        scatter_wait(2, 0)
        scatter_wait(3, 1)

        plsc.subcore_barrier()

        @pl.loop(0, rows_per_tile, step=_CH)
        def _(z):
            r0 = row0 + z
            pltpu.sync_copy(acc.at[pl.ds(r0, _CH)],
                            out_hbm.at[cid, pl.ds(r0, _CH)])

    return k(t, map_arr, dinv, sE, dE, wE)


def _tc_matmul(x, w):
    m, kdim = x.shape
    blk = 2000

    def body(x_ref, w_ref, o_ref):
        o_ref[...] = jnp.dot(x_ref[...], w_ref[...],
                             preferred_element_type=jnp.float32)

    return pl.pallas_call(
        body,
        grid=(m // blk,),
        in_specs=[
            pl.BlockSpec((blk, kdim), lambda i: (i, 0)),
            pl.BlockSpec((kdim, w.shape[1]), lambda i: (0, 0)),
        ],
        out_specs=pl.BlockSpec((blk, w.shape[1]), lambda i: (i, 0)),
        out_shape=jax.ShapeDtypeStruct((m, w.shape[1]), jnp.float32),
    )(x, w)


def _tc_dinv(degp):
    n = degp.shape[1]

    def body(p_ref, o_ref):
        deg = p_ref[0:1, :] + p_ref[1:2, :]
        o_ref[...] = jnp.where(deg > 0.0, lax.rsqrt(jnp.maximum(deg, 1e-30)),
                               0.0)

    return pl.pallas_call(
        body,
        out_shape=jax.ShapeDtypeStruct((1, n), jnp.float32),
    )(degp)


def _tc_hidden(p, dinv2d, b1, w2):
    n = p.shape[1]
    blk = 2048

    def body(p_ref, dinv_ref, b_ref, w_ref, o_ref):
        h = jnp.maximum(dinv_ref[...] * (p_ref[0] + p_ref[1]) + b_ref[...],
                        0.0)
        o_ref[...] = jnp.dot(h, w_ref[...], preferred_element_type=jnp.float32)

    return pl.pallas_call(
        body,
        grid=(n // blk,),
        in_specs=[
            pl.BlockSpec((_NC, blk, _D), lambda i: (0, i, 0)),
            pl.BlockSpec((blk, 1), lambda i: (i, 0)),
            pl.BlockSpec((1, _D), lambda i: (0, 0)),
            pl.BlockSpec((_D, _D), lambda i: (0, 0)),
        ],
        out_specs=pl.BlockSpec((blk, _D), lambda i: (i, 0)),
        out_shape=jax.ShapeDtypeStruct((n, _D), jnp.float32),
    )(p, dinv2d, b1, w2)


def _tc_final(q, dinv2d, b2):
    n = q.shape[1]
    blk = 2048

    def body(q_ref, dinv_ref, b_ref, o_ref):
        o_ref[...] = dinv_ref[...] * (q_ref[0] + q_ref[1]) + b_ref[...]

    return pl.pallas_call(
        body,
        grid=(n // blk,),
        in_specs=[
            pl.BlockSpec((_NC, blk, _D), lambda i: (0, i, 0)),
            pl.BlockSpec((blk, 1), lambda i: (i, 0)),
            pl.BlockSpec((1, _D), lambda i: (0, 0)),
        ],
        out_specs=pl.BlockSpec((blk, _D), lambda i: (i, 0)),
        out_shape=jax.ShapeDtypeStruct((n, _D), jnp.float32),
    )(q, dinv2d, b2)


def kernel(label, edge_index, weight, table, W1, b1, W2, b2):
    n = label.shape[0]
    e = weight.shape[0]
    m = e + n
    n_acc = -(-n // (_NS * _CH)) * (_NS * _CH)

    align = _NW * _CHM * 4
    m_pad = -(-m // align) * align
    pad = m_pad - m
    n_chunks_deg = m_pad // (_NW * _CH)
    n_chunks_msg = m_pad // (_NW * _CHM)

    src = edge_index[0].astype(jnp.int32)
    dst = edge_index[1].astype(jnp.int32)
    loop = jnp.arange(n, dtype=jnp.int32)
    padr = jnp.arange(pad, dtype=jnp.int32) % n
    sE = jnp.concatenate([src, loop, padr])
    dE = jnp.concatenate([dst, loop, padr])
    wE = jnp.concatenate([weight.astype(jnp.float32),
                          jnp.ones((n,), jnp.float32),
                          jnp.zeros((pad,), jnp.float32)])

    degp = _sc_degree(dE, wE, n_acc, n_chunks_deg)
    t1 = _tc_matmul(table, W1)
    dinv_row = _tc_dinv(degp)
    dinv = dinv_row[0, :n]
    dinv2d_pad = dinv_row.reshape((n_acc, 1))

    p = _sc_msg_pass(t1, label.astype(jnp.int32), dinv, sE, dE, wE, n_acc,
                     n_chunks_msg)
    t2 = _tc_hidden(p, dinv2d_pad, b1.reshape((1, _D)), W2)
    q = _sc_msg_pass(t2, loop, dinv, sE, dE, wE, n_acc, n_chunks_msg)
    return _tc_final(q, dinv2d_pad, b2.reshape((1, _D)))[:n]


# speedup vs baseline: 20.3814x; 1.2905x over previous
import dataclasses
import functools

import jax
import jax.numpy as jnp
from jax import lax
from jax.experimental import pallas as pl
from jax.experimental.pallas import tpu as pltpu
from jax.experimental.pallas import tpu_sc as plsc

_NC = 2
_NS = 16
_NW = _NC * _NS
_L = 16
_CH = 128
_CHM = 64
_D = 128


def _sc_mesh():
    return plsc.VectorSubcoreMesh(core_axis_name="c", subcore_axis_name="s")


def _sc_params():
    cp = pltpu.CompilerParams()
    if "needs_layout_passes" in pltpu.CompilerParams.__dataclass_fields__:
        cp = dataclasses.replace(cp, needs_layout_passes=False)
    return cp


def _sc_degree(dE, wE, n_acc, n_chunks):
    m_pad = dE.shape[0]
    per_w = m_pad // _NW
    rows_per_tile = n_acc // _NS
    zsteps = rows_per_tile // _CH

    @functools.partial(
        pl.kernel,
        out_type=jax.ShapeDtypeStruct((_NC, n_acc), jnp.float32),
        mesh=_sc_mesh(),
        scratch_types=[
            pltpu.VMEM((_CH,), jnp.int32),
            pltpu.VMEM((_CH,), jnp.float32),
            pltpu.VMEM((_CH,), jnp.float32),
            pltpu.VMEM_SHARED((n_acc,), jnp.float32),
        ],
        compiler_params=_sc_params(),
    )
    def k(d_hbm, w_hbm, out_hbm, d_v, w_v, z_v, acc):
        cid = lax.axis_index("c")
        sid = lax.axis_index("s")
        wid = cid * _NS + sid
        zero16 = jnp.zeros((_L,), jnp.float32)

        @pl.loop(0, _CH, step=_L)
        def _(r):
            z_v[pl.ds(r, _L)] = zero16

        row0 = sid * rows_per_tile

        @pl.loop(0, zsteps)
        def _(i):
            pltpu.sync_copy(z_v, acc.at[pl.ds(row0 + i * _CH, _CH)])

        plsc.subcore_barrier()

        base = wid * per_w

        @pl.loop(0, n_chunks)
        def _(ci):
            off = base + ci * _CH
            pltpu.sync_copy(d_hbm.at[pl.ds(off, _CH)], d_v)
            pltpu.sync_copy(w_hbm.at[pl.ds(off, _CH)], w_v)
            pltpu.sync_copy(w_v, acc.at[d_v], add=True)

        plsc.subcore_barrier()

        @pl.loop(0, zsteps)
        def _(i):
            r0 = row0 + i * _CH
            pltpu.sync_copy(acc.at[pl.ds(r0, _CH)],
                            out_hbm.at[cid, pl.ds(r0, _CH)])

    return k(dE, wE)


def _sc_msg_pass(t, map_arr, dinv, sE, dE, wE, n_acc, n_chunks):
    m_pad = sE.shape[0]
    per_w = m_pad // _NW
    n = map_arr.shape[0]
    rows_per_tile = n_acc // _NS
    assert n_chunks % 4 == 0 and n_chunks >= 8

    @functools.partial(
        pl.kernel,
        out_type=jax.ShapeDtypeStruct((_NC, n_acc, _D), jnp.float32),
        mesh=_sc_mesh(),
        scratch_types=(
            [pltpu.VMEM((n,), jnp.int32),
             pltpu.VMEM((n,), jnp.float32)]
            + [pltpu.VMEM((_CHM,), jnp.int32) for _ in range(4)]
            + [pltpu.VMEM((_CHM,), jnp.int32) for _ in range(4)]
            + [pltpu.VMEM((_CHM,), jnp.float32) for _ in range(4)]
            + [pltpu.VMEM((_CHM,), jnp.int32) for _ in range(4)]
            + [pltpu.VMEM((_CHM,), jnp.float32) for _ in range(4)]
            + [pltpu.VMEM((_CHM, _D), jnp.float32) for _ in range(2)]
            + [pltpu.VMEM_SHARED((n_acc, _D), jnp.float32)]
            + [pltpu.SemaphoreType.DMA for _ in range(8)]
        ),
        compiler_params=_sc_params(),
    )
    def k(t_hbm, map_hbm, dinv_hbm, s_hbm, d_hbm, w_hbm, out_hbm, *sc):
        map_v, dinv_v = sc[0], sc[1]
        svs, dvs, wvs = sc[2:6], sc[6:10], sc[10:14]
        lvs, cvs = sc[14:18], sc[18:22]
        rows = sc[22:24]
        acc = sc[24]
        isems, gsems, ssems = sc[25:29], sc[29:31], sc[31:33]

        cid = lax.axis_index("c")
        sid = lax.axis_index("s")
        wid = cid * _NS + sid
        base = wid * per_w
        row0 = sid * rows_per_tile

        pltpu.sync_copy(map_hbm, map_v)
        pltpu.sync_copy(dinv_hbm, dinv_v)

        zero16 = jnp.zeros((_L,), jnp.float32)

        @pl.loop(0, _CHM)
        def _(r):
            @pl.loop(0, _D, step=_L)
            def _(cc):
                rows[0][r, pl.ds(cc, _L)] = zero16

        @pl.loop(0, rows_per_tile, step=_CHM)
        def _(z):
            pltpu.sync_copy(rows[0], acc.at[pl.ds(row0 + z, _CHM)])

        plsc.subcore_barrier()

        def idx_start(i, q):
            off = base + i * _CHM
            pltpu.async_copy(s_hbm.at[pl.ds(off, _CHM)], svs[q], isems[q])
            pltpu.async_copy(d_hbm.at[pl.ds(off, _CHM)], dvs[q], isems[q])
            pltpu.async_copy(w_hbm.at[pl.ds(off, _CHM)], wvs[q], isems[q])

        def idx_wait(i, q):
            off = base + i * _CHM
            pltpu.make_async_copy(s_hbm.at[pl.ds(off, _CHM)], svs[q],
                                  isems[q]).wait()
            pltpu.make_async_copy(d_hbm.at[pl.ds(off, _CHM)], dvs[q],
                                  isems[q]).wait()
            pltpu.make_async_copy(w_hbm.at[pl.ds(off, _CHM)], wvs[q],
                                  isems[q]).wait()

        def compute_lc(q):
            @plsc.parallel_loop(0, _CHM, step=_L, unroll=2)
            def _(kk):
                s16 = svs[q][pl.ds(kk, _L)]
                lvs[q][pl.ds(kk, _L)] = plsc.load_gather(map_v, [s16])
                cvs[q][pl.ds(kk, _L)] = wvs[q][pl.ds(kk, _L)] * (
                    plsc.load_gather(dinv_v, [s16]))

        def gather_start(q, b):
            pltpu.async_copy(t_hbm.at[lvs[q]], rows[b], gsems[b])

        def gather_wait(q, b):
            pltpu.make_async_copy(t_hbm.at[lvs[q]], rows[b], gsems[b]).wait()

        def scatter_start(q, b):
            pltpu.async_copy(rows[b], acc.at[dvs[q]], ssems[b], add=True)

        def scatter_wait(q, b):
            pltpu.make_async_copy(rows[b], acc.at[dvs[q]], ssems[b]).wait()

        idx_start(0, 0)
        idx_start(1, 1)
        idx_start(2, 2)
        idx_wait(0, 0)
        compute_lc(0)
        gather_start(0, 0)

        @pl.loop(0, n_chunks, step=4)
        def _(g):
            for j in range(4):
                q, b = j, j & 1
                q2, b2 = (j + 1) & 3, 1 - (j & 1)
                i = g + j

                gather_wait(q, b)

                @pl.when(i + 1 < n_chunks)
                def _():
                    @pl.when(i >= 1)
                    def _():
                        scatter_wait((j + 3) & 3, b2)

                    idx_wait(i + 1, q2)
                    compute_lc(q2)
                    gather_start(q2, b2)

                    @pl.when(i + 3 < n_chunks)
                    def _():
                        idx_start(i + 3, (j + 3) & 3)

                @plsc.parallel_loop(0, _CHM, unroll=4)
                def _(e):
                    cb = plsc.load_gather(
                        cvs[q], [jnp.full((_L,), 0, jnp.int32) + e])
                    for jj in range(_D // _L):
                        sl = pl.ds(jj * _L, _L)
                        rows[b][e, sl] = rows[b][e, sl] * cb

                scatter_start(q, b)

        scatter_wait(2, 0)
        scatter_wait(3, 1)

        plsc.subcore_barrier()

        @pl.loop(0, rows_per_tile, step=_CH)
        def _(z):
            r0 = row0 + z
            pltpu.sync_copy(acc.at[pl.ds(r0, _CH)],
                            out_hbm.at[cid, pl.ds(r0, _CH)])

    return k(t, map_arr, dinv, sE, dE, wE)


def _sc_msg_pass_pre(t, sE, dE, wE, n_acc, n_chunks):
    m_pad = sE.shape[0]
    per_w = m_pad // _NW
    rows_per_tile = n_acc // _NS
    assert n_chunks % 4 == 0 and n_chunks >= 8

    @functools.partial(
        pl.kernel,
        out_type=jax.ShapeDtypeStruct((_NC, n_acc, _D), jnp.float32),
        mesh=_sc_mesh(),
        scratch_types=(
            [pltpu.VMEM((_CH,), jnp.int32) for _ in range(4)]
            + [pltpu.VMEM((_CH,), jnp.int32) for _ in range(4)]
            + [pltpu.VMEM((_CH,), jnp.float32) for _ in range(4)]
            + [pltpu.VMEM((_CH, _D), jnp.float32) for _ in range(2)]
            + [pltpu.VMEM_SHARED((n_acc, _D), jnp.float32)]
            + [pltpu.SemaphoreType.DMA for _ in range(8)]
        ),
        compiler_params=_sc_params(),
    )
    def k(t_hbm, s_hbm, d_hbm, w_hbm, out_hbm, *sc):
        svs, dvs, wvs = sc[0:4], sc[4:8], sc[8:12]
        rows = sc[12:14]
        acc = sc[14]
        isems, gsems, ssems = sc[15:19], sc[19:21], sc[21:23]

        cid = lax.axis_index("c")
        sid = lax.axis_index("s")
        wid = cid * _NS + sid
        base = wid * per_w
        row0 = sid * rows_per_tile

        zero16 = jnp.zeros((_L,), jnp.float32)

        @pl.loop(0, _CH)
        def _(r):
            @pl.loop(0, _D, step=_L)
            def _(cc):
                rows[0][r, pl.ds(cc, _L)] = zero16

        @pl.loop(0, rows_per_tile, step=_CH)
        def _(z):
            pltpu.sync_copy(rows[0], acc.at[pl.ds(row0 + z, _CH)])

        plsc.subcore_barrier()

        def idx_start(i, q):
            off = base + i * _CH
            pltpu.async_copy(s_hbm.at[pl.ds(off, _CH)], svs[q], isems[q])
            pltpu.async_copy(d_hbm.at[pl.ds(off, _CH)], dvs[q], isems[q])
            pltpu.async_copy(w_hbm.at[pl.ds(off, _CH)], wvs[q], isems[q])

        def idx_wait(i, q):
            off = base + i * _CH
            pltpu.make_async_copy(s_hbm.at[pl.ds(off, _CH)], svs[q],
                                  isems[q]).wait()
            pltpu.make_async_copy(d_hbm.at[pl.ds(off, _CH)], dvs[q],
                                  isems[q]).wait()
            pltpu.make_async_copy(w_hbm.at[pl.ds(off, _CH)], wvs[q],
                                  isems[q]).wait()

        def gather_start(q, b):
            pltpu.async_copy(t_hbm.at[svs[q]], rows[b], gsems[b])

        def gather_wait(q, b):
            pltpu.make_async_copy(t_hbm.at[svs[q]], rows[b], gsems[b]).wait()

        def scatter_start(q, b):
            pltpu.async_copy(rows[b], acc.at[dvs[q]], ssems[b], add=True)

        def scatter_wait(q, b):
            pltpu.make_async_copy(rows[b], acc.at[dvs[q]], ssems[b]).wait()

        idx_start(0, 0)
        idx_start(1, 1)
        idx_start(2, 2)
        idx_wait(0, 0)
        gather_start(0, 0)

        @pl.loop(0, n_chunks, step=4)
        def _(g):
            for j in range(4):
                q, b = j, j & 1
                q2, b2 = (j + 1) & 3, 1 - (j & 1)
                i = g + j

                gather_wait(q, b)

                @pl.when(i + 1 < n_chunks)
                def _():
                    @pl.when(i >= 1)
                    def _():
                        scatter_wait((j + 3) & 3, b2)

                    idx_wait(i + 1, q2)
                    gather_start(q2, b2)

                    @pl.when(i + 3 < n_chunks)
                    def _():
                        idx_start(i + 3, (j + 3) & 3)

                @plsc.parallel_loop(0, _CH, unroll=4)
                def _(e):
                    cb = plsc.load_gather(
                        wvs[q], [jnp.full((_L,), 0, jnp.int32) + e])
                    for jj in range(_D // _L):
                        sl = pl.ds(jj * _L, _L)
                        rows[b][e, sl] = rows[b][e, sl] * cb

                scatter_start(q, b)

        scatter_wait(2, 0)
        scatter_wait(3, 1)

        plsc.subcore_barrier()

        @pl.loop(0, rows_per_tile, step=_CH)
        def _(z):
            r0 = row0 + z
            pltpu.sync_copy(acc.at[pl.ds(r0, _CH)],
                            out_hbm.at[cid, pl.ds(r0, _CH)])

    return k(t, sE, dE, wE)


def _tc_matmul(x, w):
    m, kdim = x.shape
    blk = 2000

    def body(x_ref, w_ref, o_ref):
        o_ref[...] = jnp.dot(x_ref[...], w_ref[...],
                             preferred_element_type=jnp.float32)

    return pl.pallas_call(
        body,
        grid=(m // blk,),
        in_specs=[
            pl.BlockSpec((blk, kdim), lambda i: (i, 0)),
            pl.BlockSpec((kdim, w.shape[1]), lambda i: (0, 0)),
        ],
        out_specs=pl.BlockSpec((blk, w.shape[1]), lambda i: (i, 0)),
        out_shape=jax.ShapeDtypeStruct((m, w.shape[1]), jnp.float32),
    )(x, w)


def _tc_dinv(degp):
    n = degp.shape[1]

    def body(p_ref, o_ref):
        deg = p_ref[0:1, :] + p_ref[1:2, :]
        o_ref[...] = jnp.where(deg > 0.0, lax.rsqrt(jnp.maximum(deg, 1e-30)),
                               0.0)

    return pl.pallas_call(
        body,
        out_shape=jax.ShapeDtypeStruct((1, n), jnp.float32),
    )(degp)


def _tc_hidden(p, dinv2d, b1, w2):
    n = p.shape[1]
    blk = 2048

    def body(p_ref, dinv_ref, b_ref, w_ref, o_ref):
        h = jnp.maximum(dinv_ref[...] * (p_ref[0] + p_ref[1]) + b_ref[...],
                        0.0)
        o_ref[...] = dinv_ref[...] * jnp.dot(
            h, w_ref[...], preferred_element_type=jnp.float32)

    return pl.pallas_call(
        body,
        grid=(n // blk,),
        in_specs=[
            pl.BlockSpec((_NC, blk, _D), lambda i: (0, i, 0)),
            pl.BlockSpec((blk, 1), lambda i: (i, 0)),
            pl.BlockSpec((1, _D), lambda i: (0, 0)),
            pl.BlockSpec((_D, _D), lambda i: (0, 0)),
        ],
        out_specs=pl.BlockSpec((blk, _D), lambda i: (i, 0)),
        out_shape=jax.ShapeDtypeStruct((n, _D), jnp.float32),
    )(p, dinv2d, b1, w2)


def _tc_final(q, dinv2d, b2):
    n = q.shape[1]
    blk = 2048

    def body(q_ref, dinv_ref, b_ref, o_ref):
        o_ref[...] = dinv_ref[...] * (q_ref[0] + q_ref[1]) + b_ref[...]

    return pl.pallas_call(
        body,
        grid=(n // blk,),
        in_specs=[
            pl.BlockSpec((_NC, blk, _D), lambda i: (0, i, 0)),
            pl.BlockSpec((blk, 1), lambda i: (i, 0)),
            pl.BlockSpec((1, _D), lambda i: (0, 0)),
        ],
        out_specs=pl.BlockSpec((blk, _D), lambda i: (i, 0)),
        out_shape=jax.ShapeDtypeStruct((n, _D), jnp.float32),
    )(q, dinv2d, b2)


def kernel(label, edge_index, weight, table, W1, b1, W2, b2):
    n = label.shape[0]
    e = weight.shape[0]
    m = e + n
    n_acc = -(-n // (_NS * _CH)) * (_NS * _CH)

    align = _NW * _CH * 4
    m_pad = -(-m // align) * align
    pad = m_pad - m
    n_chunks_big = m_pad // (_NW * _CH)
    n_chunks_msg = m_pad // (_NW * _CHM)

    src = edge_index[0].astype(jnp.int32)
    dst = edge_index[1].astype(jnp.int32)
    loop = jnp.arange(n, dtype=jnp.int32)
    padr = jnp.arange(pad, dtype=jnp.int32) % n
    sE = jnp.concatenate([src, loop, padr])
    dE = jnp.concatenate([dst, loop, padr])
    wE = jnp.concatenate([weight.astype(jnp.float32),
                          jnp.ones((n,), jnp.float32),
                          jnp.zeros((pad,), jnp.float32)])

    degp = _sc_degree(dE, wE, n_acc, n_chunks_big)
    t1 = _tc_matmul(table, W1)
    dinv_row = _tc_dinv(degp)
    dinv = dinv_row[0, :n]
    dinv2d_pad = dinv_row.reshape((n_acc, 1))

    p = _sc_msg_pass(t1, label.astype(jnp.int32), dinv, sE, dE, wE, n_acc,
                     n_chunks_msg)
    t2 = _tc_hidden(p, dinv2d_pad, b1.reshape((1, _D)), W2)
    q = _sc_msg_pass_pre(t2, sE, dE, wE, n_acc, n_chunks_big)
    return _tc_final(q, dinv2d_pad, b2.reshape((1, _D)))[:n]


# speedup vs baseline: 25.4549x; 1.2489x over previous
import dataclasses
import functools

import jax
import jax.numpy as jnp
from jax import lax
from jax.experimental import pallas as pl
from jax.experimental.pallas import tpu as pltpu
from jax.experimental.pallas import tpu_sc as plsc

_NC = 2
_NS = 16
_NW = _NC * _NS
_L = 16
_CH = 128
_CHM = 64
_D = 128


def _sc_mesh():
    return plsc.VectorSubcoreMesh(core_axis_name="c", subcore_axis_name="s")


def _sc_params():
    cp = pltpu.CompilerParams()
    if "needs_layout_passes" in pltpu.CompilerParams.__dataclass_fields__:
        cp = dataclasses.replace(cp, needs_layout_passes=False)
    return cp


def _sc_degree(dE, wE, n_acc, n_chunks):
    m_pad = dE.shape[0]
    per_w = m_pad // _NW
    rows_per_tile = n_acc // _NS
    zsteps = rows_per_tile // _CH

    assert n_chunks % 4 == 0 and n_chunks >= 8

    @functools.partial(
        pl.kernel,
        out_type=jax.ShapeDtypeStruct((_NC, n_acc), jnp.float32),
        mesh=_sc_mesh(),
        scratch_types=(
            [pltpu.VMEM((_CH,), jnp.int32) for _ in range(4)]
            + [pltpu.VMEM((_CH,), jnp.float32) for _ in range(4)]
            + [pltpu.VMEM((_CH,), jnp.int32) for _ in range(2)]
            + [pltpu.VMEM((_CH,), jnp.float32) for _ in range(2)]
            + [pltpu.VMEM_SHARED((n_acc,), jnp.float32)]
            + [pltpu.SemaphoreType.DMA for _ in range(6)]
        ),
        compiler_params=_sc_params(),
    )
    def k(d_hbm, w_hbm, out_hbm, *sc):
        dvs, wvs = sc[0:4], sc[4:8]
        dss, wss = sc[8:10], sc[10:12]
        acc = sc[12]
        isems, ssems = sc[13:17], sc[17:19]

        cid = lax.axis_index("c")
        sid = lax.axis_index("s")
        wid = cid * _NS + sid
        zero16 = jnp.zeros((_L,), jnp.float32)
        row0 = sid * rows_per_tile
        base = wid * per_w

        @pl.loop(0, _CH, step=_L)
        def _(r):
            wss[0][pl.ds(r, _L)] = zero16

        @pl.loop(0, zsteps)
        def _(i):
            pltpu.sync_copy(wss[0], acc.at[pl.ds(row0 + i * _CH, _CH)])

        plsc.subcore_barrier()

        def idx_start(i, q):
            off = base + i * _CH
            pltpu.async_copy(d_hbm.at[pl.ds(off, _CH)], dvs[q], isems[q])
            pltpu.async_copy(w_hbm.at[pl.ds(off, _CH)], wvs[q], isems[q])

        def idx_wait(i, q):
            off = base + i * _CH
            pltpu.make_async_copy(d_hbm.at[pl.ds(off, _CH)], dvs[q],
                                  isems[q]).wait()
            pltpu.make_async_copy(w_hbm.at[pl.ds(off, _CH)], wvs[q],
                                  isems[q]).wait()

        for j in range(4):
            idx_start(j, j)

        @pl.loop(0, n_chunks, step=4)
        def _(g):
            for j in range(4):
                q, r = j, j & 1
                i = g + j
                idx_wait(i, q)

                @pl.when(i >= 2)
                def _():
                    pltpu.make_async_copy(wss[r], acc.at[dss[r]],
                                          ssems[r]).wait()

                @plsc.parallel_loop(0, _CH, step=_L, unroll=2)
                def _(kk):
                    dss[r][pl.ds(kk, _L)] = dvs[q][pl.ds(kk, _L)]
                    wss[r][pl.ds(kk, _L)] = wvs[q][pl.ds(kk, _L)]

                pltpu.async_copy(wss[r], acc.at[dss[r]], ssems[r], add=True)

                @pl.when(i + 4 < n_chunks)
                def _():
                    idx_start(i + 4, q)

        pltpu.make_async_copy(wss[0], acc.at[dss[0]], ssems[0]).wait()
        pltpu.make_async_copy(wss[1], acc.at[dss[1]], ssems[1]).wait()

        plsc.subcore_barrier()

        @pl.loop(0, zsteps)
        def _(i):
            r0 = row0 + i * _CH
            pltpu.sync_copy(acc.at[pl.ds(r0, _CH)],
                            out_hbm.at[cid, pl.ds(r0, _CH)])

    return k(dE, wE)


def _sc_msg_pass(t, map_arr, dinv, sE, dE, wE, n_acc, n_chunks):
    m_pad = sE.shape[0]
    per_w = m_pad // _NW
    n = map_arr.shape[0]
    rows_per_tile = n_acc // _NS
    assert n_chunks % 4 == 0 and n_chunks >= 8

    @functools.partial(
        pl.kernel,
        out_type=jax.ShapeDtypeStruct((_NC, n_acc, _D), jnp.float32),
        mesh=_sc_mesh(),
        scratch_types=(
            [pltpu.VMEM((n,), jnp.int32),
             pltpu.VMEM((n,), jnp.float32)]
            + [pltpu.VMEM((_CHM,), jnp.int32) for _ in range(4)]
            + [pltpu.VMEM((_CHM,), jnp.int32) for _ in range(4)]
            + [pltpu.VMEM((_CHM,), jnp.float32) for _ in range(4)]
            + [pltpu.VMEM((_CHM,), jnp.int32) for _ in range(4)]
            + [pltpu.VMEM((_CHM,), jnp.float32) for _ in range(4)]
            + [pltpu.VMEM((_CHM, _D), jnp.float32) for _ in range(2)]
            + [pltpu.VMEM_SHARED((n_acc, _D), jnp.float32)]
            + [pltpu.SemaphoreType.DMA for _ in range(8)]
        ),
        compiler_params=_sc_params(),
    )
    def k(t_hbm, map_hbm, dinv_hbm, s_hbm, d_hbm, w_hbm, out_hbm, *sc):
        map_v, dinv_v = sc[0], sc[1]
        svs, dvs, wvs = sc[2:6], sc[6:10], sc[10:14]
        lvs, cvs = sc[14:18], sc[18:22]
        rows = sc[22:24]
        acc = sc[24]
        isems, gsems, ssems = sc[25:29], sc[29:31], sc[31:33]

        cid = lax.axis_index("c")
        sid = lax.axis_index("s")
        wid = cid * _NS + sid
        base = wid * per_w
        row0 = sid * rows_per_tile

        pltpu.sync_copy(map_hbm, map_v)
        pltpu.sync_copy(dinv_hbm, dinv_v)

        zero16 = jnp.zeros((_L,), jnp.float32)

        @pl.loop(0, _CHM)
        def _(r):
            @pl.loop(0, _D, step=_L)
            def _(cc):
                rows[0][r, pl.ds(cc, _L)] = zero16

        @pl.loop(0, rows_per_tile, step=_CHM)
        def _(z):
            pltpu.sync_copy(rows[0], acc.at[pl.ds(row0 + z, _CHM)])

        plsc.subcore_barrier()

        def idx_start(i, q):
            off = base + i * _CHM
            pltpu.async_copy(s_hbm.at[pl.ds(off, _CHM)], svs[q], isems[q])
            pltpu.async_copy(d_hbm.at[pl.ds(off, _CHM)], dvs[q], isems[q])
            pltpu.async_copy(w_hbm.at[pl.ds(off, _CHM)], wvs[q], isems[q])

        def idx_wait(i, q):
            off = base + i * _CHM
            pltpu.make_async_copy(s_hbm.at[pl.ds(off, _CHM)], svs[q],
                                  isems[q]).wait()
            pltpu.make_async_copy(d_hbm.at[pl.ds(off, _CHM)], dvs[q],
                                  isems[q]).wait()
            pltpu.make_async_copy(w_hbm.at[pl.ds(off, _CHM)], wvs[q],
                                  isems[q]).wait()

        def compute_lc(q):
            @plsc.parallel_loop(0, _CHM, step=_L, unroll=2)
            def _(kk):
                s16 = svs[q][pl.ds(kk, _L)]
                lvs[q][pl.ds(kk, _L)] = plsc.load_gather(map_v, [s16])
                cvs[q][pl.ds(kk, _L)] = wvs[q][pl.ds(kk, _L)] * (
                    plsc.load_gather(dinv_v, [s16]))

        def gather_start(q, b):
            pltpu.async_copy(t_hbm.at[lvs[q]], rows[b], gsems[b])

        def gather_wait(q, b):
            pltpu.make_async_copy(t_hbm.at[lvs[q]], rows[b], gsems[b]).wait()

        def scatter_start(q, b):
            pltpu.async_copy(rows[b], acc.at[dvs[q]], ssems[b], add=True)

        def scatter_wait(q, b):
            pltpu.make_async_copy(rows[b], acc.at[dvs[q]], ssems[b]).wait()

        idx_start(0, 0)
        idx_start(1, 1)
        idx_start(2, 2)
        idx_wait(0, 0)
        compute_lc(0)
        gather_start(0, 0)

        @pl.loop(0, n_chunks, step=4)
        def _(g):
            for j in range(4):
                q, b = j, j & 1
                q2, b2 = (j + 1) & 3, 1 - (j & 1)
                i = g + j

                gather_wait(q, b)

                @pl.when(i + 1 < n_chunks)
                def _():
                    @pl.when(i >= 1)
                    def _():
                        scatter_wait((j + 3) & 3, b2)

                    idx_wait(i + 1, q2)
                    compute_lc(q2)
                    gather_start(q2, b2)

                    @pl.when(i + 3 < n_chunks)
                    def _():
                        idx_start(i + 3, (j + 3) & 3)

                @plsc.parallel_loop(0, _CHM, unroll=4)
                def _(e):
                    cb = plsc.load_gather(
                        cvs[q], [jnp.full((_L,), 0, jnp.int32) + e])
                    for jj in range(_D // _L):
                        sl = pl.ds(jj * _L, _L)
                        rows[b][e, sl] = rows[b][e, sl] * cb

                scatter_start(q, b)

        scatter_wait(2, 0)
        scatter_wait(3, 1)

        plsc.subcore_barrier()

        @pl.loop(0, rows_per_tile, step=_CH)
        def _(z):
            r0 = row0 + z
            pltpu.sync_copy(acc.at[pl.ds(r0, _CH)],
                            out_hbm.at[cid, pl.ds(r0, _CH)])

    return k(t, map_arr, dinv, sE, dE, wE)


def _sc_gather_rows(t, idx, n_acc):
    rows_per_tile = n_acc // _NS
    nch = rows_per_tile // _CHM

    @functools.partial(
        pl.kernel,
        out_type=jax.ShapeDtypeStruct((n_acc, _D), jnp.float32),
        mesh=_sc_mesh(),
        scratch_types=[
            pltpu.VMEM((_CHM,), jnp.int32),
            pltpu.VMEM((_CHM, _D), jnp.float32),
            pltpu.SemaphoreType.DMA,
        ],
        compiler_params=_sc_params(),
    )
    def k(t_hbm, i_hbm, out_hbm, i_v, r_v, sem):
        cid = lax.axis_index("c")
        sid = lax.axis_index("s")
        half = rows_per_tile // 2
        base = sid * rows_per_tile + cid * half

        @pl.loop(0, half, step=_CHM)
        def _(z):
            off = base + z
            pltpu.sync_copy(i_hbm.at[pl.ds(off, _CHM)], i_v)
            pltpu.async_copy(t_hbm.at[i_v], r_v, sem).wait()
            pltpu.sync_copy(r_v, out_hbm.at[pl.ds(off, _CHM)])

    return k(t, idx)


def _sc_msg_pass_pre(t, sE, dE, wE, n_acc, n_chunks):
    m_pad = sE.shape[0]
    per_w = m_pad // _NW
    rows_per_tile = n_acc // _NS
    assert n_chunks % 4 == 0 and n_chunks >= 8

    @functools.partial(
        pl.kernel,
        out_type=jax.ShapeDtypeStruct((_NC, n_acc, _D), jnp.float32),
        mesh=_sc_mesh(),
        scratch_types=(
            [pltpu.VMEM((_CH,), jnp.int32) for _ in range(4)]
            + [pltpu.VMEM((_CH,), jnp.int32) for _ in range(4)]
            + [pltpu.VMEM((_CH,), jnp.float32) for _ in range(4)]
            + [pltpu.VMEM((_CH, _D), jnp.float32) for _ in range(2)]
            + [pltpu.VMEM_SHARED((n_acc, _D), jnp.float32)]
            + [pltpu.SemaphoreType.DMA for _ in range(8)]
        ),
        compiler_params=_sc_params(),
    )
    def k(t_hbm, s_hbm, d_hbm, w_hbm, out_hbm, *sc):
        svs, dvs, wvs = sc[0:4], sc[4:8], sc[8:12]
        rows = sc[12:14]
        acc = sc[14]
        isems, gsems, ssems = sc[15:19], sc[19:21], sc[21:23]

        cid = lax.axis_index("c")
        sid = lax.axis_index("s")
        wid = cid * _NS + sid
        base = wid * per_w
        row0 = sid * rows_per_tile

        zero16 = jnp.zeros((_L,), jnp.float32)

        @pl.loop(0, _CH)
        def _(r):
            @pl.loop(0, _D, step=_L)
            def _(cc):
                rows[0][r, pl.ds(cc, _L)] = zero16

        @pl.loop(0, rows_per_tile, step=_CH)
        def _(z):
            pltpu.sync_copy(rows[0], acc.at[pl.ds(row0 + z, _CH)])

        plsc.subcore_barrier()

        def idx_start(i, q):
            off = base + i * _CH
            pltpu.async_copy(s_hbm.at[pl.ds(off, _CH)], svs[q], isems[q])
            pltpu.async_copy(d_hbm.at[pl.ds(off, _CH)], dvs[q], isems[q])
            pltpu.async_copy(w_hbm.at[pl.ds(off, _CH)], wvs[q], isems[q])

        def idx_wait(i, q):
            off = base + i * _CH
            pltpu.make_async_copy(s_hbm.at[pl.ds(off, _CH)], svs[q],
                                  isems[q]).wait()
            pltpu.make_async_copy(d_hbm.at[pl.ds(off, _CH)], dvs[q],
                                  isems[q]).wait()
            pltpu.make_async_copy(w_hbm.at[pl.ds(off, _CH)], wvs[q],
                                  isems[q]).wait()

        def gather_start(q, b):
            pltpu.async_copy(t_hbm.at[svs[q]], rows[b], gsems[b])

        def gather_wait(q, b):
            pltpu.make_async_copy(t_hbm.at[svs[q]], rows[b], gsems[b]).wait()

        def scatter_start(q, b):
            pltpu.async_copy(rows[b], acc.at[dvs[q]], ssems[b], add=True)

        def scatter_wait(q, b):
            pltpu.make_async_copy(rows[b], acc.at[dvs[q]], ssems[b]).wait()

        idx_start(0, 0)
        idx_start(1, 1)
        idx_start(2, 2)
        idx_wait(0, 0)
        gather_start(0, 0)

        @pl.loop(0, n_chunks, step=4)
        def _(g):
            for j in range(4):
                q, b = j, j & 1
                q2, b2 = (j + 1) & 3, 1 - (j & 1)
                i = g + j

                gather_wait(q, b)

                @pl.when(i + 1 < n_chunks)
                def _():
                    @pl.when(i >= 1)
                    def _():
                        scatter_wait((j + 3) & 3, b2)

                    idx_wait(i + 1, q2)
                    gather_start(q2, b2)

                    @pl.when(i + 3 < n_chunks)
                    def _():
                        idx_start(i + 3, (j + 3) & 3)

                @plsc.parallel_loop(0, _CH, unroll=4)
                def _(e):
                    cb = plsc.load_gather(
                        wvs[q], [jnp.full((_L,), 0, jnp.int32) + e])
                    for jj in range(_D // _L):
                        sl = pl.ds(jj * _L, _L)
                        rows[b][e, sl] = rows[b][e, sl] * cb

                scatter_start(q, b)

        scatter_wait(2, 0)
        scatter_wait(3, 1)

        plsc.subcore_barrier()

        @pl.loop(0, rows_per_tile, step=_CH)
        def _(z):
            r0 = row0 + z
            pltpu.sync_copy(acc.at[pl.ds(r0, _CH)],
                            out_hbm.at[cid, pl.ds(r0, _CH)])

    return k(t, sE, dE, wE)


def _tc_matmul(x, w):
    m, kdim = x.shape
    blk = 2000

    def body(x_ref, w_ref, o_ref):
        o_ref[...] = jnp.dot(x_ref[...], w_ref[...],
                             preferred_element_type=jnp.float32)

    return pl.pallas_call(
        body,
        grid=(m // blk,),
        in_specs=[
            pl.BlockSpec((blk, kdim), lambda i: (i, 0)),
            pl.BlockSpec((kdim, w.shape[1]), lambda i: (0, 0)),
        ],
        out_specs=pl.BlockSpec((blk, w.shape[1]), lambda i: (i, 0)),
        out_shape=jax.ShapeDtypeStruct((m, w.shape[1]), jnp.float32),
    )(x, w)


def _tc_dinv(degp):
    n = degp.shape[1]

    def body(p_ref, o_ref):
        deg = p_ref[0:1, :] + p_ref[1:2, :]
        o_ref[...] = jnp.where(deg > 0.0, lax.rsqrt(jnp.maximum(deg, 1e-30)),
                               0.0)

    return pl.pallas_call(
        body,
        out_shape=jax.ShapeDtypeStruct((1, n), jnp.float32),
    )(degp)


def _tc_scale_rows(x, dinv2d):
    n = x.shape[0]
    blk = 2048

    def body(x_ref, dinv_ref, o_ref):
        o_ref[...] = dinv_ref[...] * x_ref[...]

    return pl.pallas_call(
        body,
        grid=(n // blk,),
        in_specs=[
            pl.BlockSpec((blk, _D), lambda i: (i, 0)),
            pl.BlockSpec((blk, 1), lambda i: (i, 0)),
        ],
        out_specs=pl.BlockSpec((blk, _D), lambda i: (i, 0)),
        out_shape=jax.ShapeDtypeStruct((n, _D), jnp.float32),
    )(x, dinv2d)


def _tc_hidden(p, dinv2d, b1, w2):
    n = p.shape[1]
    blk = 2048

    def body(p_ref, dinv_ref, b_ref, w_ref, o_ref):
        h = jnp.maximum(dinv_ref[...] * (p_ref[0] + p_ref[1]) + b_ref[...],
                        0.0)
        o_ref[...] = dinv_ref[...] * jnp.dot(
            h, w_ref[...], preferred_element_type=jnp.float32)

    return pl.pallas_call(
        body,
        grid=(n // blk,),
        in_specs=[
            pl.BlockSpec((_NC, blk, _D), lambda i: (0, i, 0)),
            pl.BlockSpec((blk, 1), lambda i: (i, 0)),
            pl.BlockSpec((1, _D), lambda i: (0, 0)),
            pl.BlockSpec((_D, _D), lambda i: (0, 0)),
        ],
        out_specs=pl.BlockSpec((blk, _D), lambda i: (i, 0)),
        out_shape=jax.ShapeDtypeStruct((n, _D), jnp.float32),
    )(p, dinv2d, b1, w2)


def _tc_final(q, dinv2d, b2):
    n = q.shape[1]
    blk = 2048

    def body(q_ref, dinv_ref, b_ref, o_ref):
        o_ref[...] = dinv_ref[...] * (q_ref[0] + q_ref[1]) + b_ref[...]

    return pl.pallas_call(
        body,
        grid=(n // blk,),
        in_specs=[
            pl.BlockSpec((_NC, blk, _D), lambda i: (0, i, 0)),
            pl.BlockSpec((blk, 1), lambda i: (i, 0)),
            pl.BlockSpec((1, _D), lambda i: (0, 0)),
        ],
        out_specs=pl.BlockSpec((blk, _D), lambda i: (i, 0)),
        out_shape=jax.ShapeDtypeStruct((n, _D), jnp.float32),
    )(q, dinv2d, b2)


def kernel(label, edge_index, weight, table, W1, b1, W2, b2):
    n = label.shape[0]
    e = weight.shape[0]
    m = e + n
    n_acc = -(-n // (_NS * _CH)) * (_NS * _CH)

    align = _NW * _CH * 4
    m_pad = -(-m // align) * align
    pad = m_pad - m
    n_chunks_big = m_pad // (_NW * _CH)
    n_chunks_msg = m_pad // (_NW * _CHM)

    src = edge_index[0].astype(jnp.int32)
    dst = edge_index[1].astype(jnp.int32)
    loop = jnp.arange(n, dtype=jnp.int32)
    padr = jnp.arange(pad, dtype=jnp.int32) % n
    sE = jnp.concatenate([src, loop, padr])
    dE = jnp.concatenate([dst, loop, padr])
    wE = jnp.concatenate([weight.astype(jnp.float32),
                          jnp.ones((n,), jnp.float32),
                          jnp.zeros((pad,), jnp.float32)])

    degp = _sc_degree(dE, wE, n_acc, n_chunks_big)
    t1 = _tc_matmul(table, W1)
    dinv_row = _tc_dinv(degp)
    dinv = dinv_row[0, :n]
    dinv2d_pad = dinv_row.reshape((n_acc, 1))

    label_pad = jnp.concatenate([label.astype(jnp.int32),
                                 jnp.zeros((n_acc - n,), jnp.int32)])
    xw1 = _sc_gather_rows(t1, label_pad, n_acc)
    y1 = _tc_scale_rows(xw1, dinv2d_pad)
    p = _sc_msg_pass_pre(y1, sE, dE, wE, n_acc, n_chunks_big)
    t2 = _tc_hidden(p, dinv2d_pad, b1.reshape((1, _D)), W2)
    q = _sc_msg_pass_pre(t2, sE, dE, wE, n_acc, n_chunks_big)
    return _tc_final(q, dinv2d_pad, b2.reshape((1, _D)))[:n]
